# baseline (speedup 1.0000x reference)
import functools

import jax
import jax.numpy as jnp
from jax import lax
from jax.experimental import pallas as pl
from jax.experimental.pallas import tpu as pltpu
from jax.experimental.pallas import tpu_sc as plsc

N = 10000
E = 320000
F_IN = 128
H = 128
HH = H // 2
OUT = 64
G = 64

NPAD = 10240
NSUB = 16
CHUNK = 128
NCHUNK = 160
EPT = CHUNK * NCHUNK
EPAD = NSUB * EPT
STRIPE = NPAD // NSUB

_mesh = plsc.VectorSubcoreMesh(core_axis_name="c", subcore_axis_name="s")
_sc_params = pltpu.CompilerParams(needs_layout_passes=False)
_sc_agg_params = pltpu.CompilerParams(use_tc_tiling_on_sc=False)


@functools.partial(
    pl.kernel,
    mesh=_mesh,
    out_type=jax.ShapeDtypeStruct((2 * NSUB, NPAD), jnp.float32),
    scratch_types=[
        pltpu.VMEM((NCHUNK // 2, CHUNK), jnp.int32),
        pltpu.VMEM((NPAD,), jnp.float32),
        pltpu.SemaphoreType.DMA,
    ],
    compiler_params=_sc_params,
)
def _sc_count(dst_hbm, out_hbm, idx_v, cnt_v, sem):
    c = lax.axis_index("c")
    s = lax.axis_index("s")
    wid = s * 2 + c
    pltpu.async_copy(dst_hbm.at[s, pl.ds(c * (NCHUNK // 2), NCHUNK // 2)],
                     idx_v, sem).wait()

    @pl.loop(0, NPAD, step=16)
    def _(i):
        cnt_v[pl.ds(i, 16)] = jnp.zeros((16,), jnp.float32)

    ones = jnp.ones((16,), jnp.float32)

    @pl.loop(0, NCHUNK // 2)
    def _(j):
        @pl.loop(0, CHUNK, step=16)
        def _(l):
            idx16 = idx_v[j, pl.ds(l, 16)]
            plsc.addupdate_scatter(cnt_v, [idx16], ones)

    pltpu.async_copy(cnt_v, out_hbm.at[wid], sem).wait()


NBUF = 8
DEPTH = NBUF // 2
NPH = 2
NCH_PH = NCHUNK // NPH


@functools.partial(
    pl.kernel,
    mesh=_mesh,
    out_type=jax.ShapeDtypeStruct((NPAD, 2, HH), jnp.float32),
    scratch_types=[
        pltpu.VMEM((NCH_PH, CHUNK), jnp.int32),
        pltpu.VMEM((NCH_PH, CHUNK), jnp.int32),
    ] + [pltpu.VMEM((CHUNK, HH), jnp.float32) for _ in range(NBUF)] + [
        pltpu.VMEM_SHARED((NPAD, HH), jnp.float32),
    ] + [pltpu.SemaphoreType.DMA for _ in range(2 * NBUF + 1)],
    compiler_params=_sc_agg_params,
)
def _sc_agg(hws_hbm, src_hbm, dst_hbm, zero_hbm, out_hbm,
            src_v, dst_v, *rest):
    bufs = rest[:NBUF]
    acc_sp = rest[NBUF]
    sg = rest[NBUF + 1:2 * NBUF + 1]
    ss = rest[2 * NBUF + 1:3 * NBUF + 1]
    saux = rest[3 * NBUF + 1]

    c = lax.axis_index("c")
    s = lax.axis_index("s")
    base = s * STRIPE

    pltpu.async_copy(zero_hbm.at[pl.ds(base, STRIPE)],
                     acc_sp.at[pl.ds(base, STRIPE)], saux).wait()
    plsc.subcore_barrier()

    @pl.loop(0, NPH)
    def _(p):
        off = p * NCH_PH
        cp_src = pltpu.async_copy(
            src_hbm.at[c, s, pl.ds(off, NCH_PH)], src_v, sg[0])
        cp_dst = pltpu.async_copy(
            dst_hbm.at[s, pl.ds(off, NCH_PH)], dst_v, sg[1])
        cp_src.wait()
        cp_dst.wait()

        for j in range(DEPTH):
            pltpu.async_copy(hws_hbm.at[src_v.at[j]], bufs[j], sg[j])

        @pl.loop(0, NCH_PH // NBUF)
        def _(k):
            j0 = NBUF * k
            for b in range(NBUF):
                j = j0 + b
                b2 = (b + DEPTH) % NBUF

                @pl.when(j >= DEPTH)
                def _():
                    pltpu.make_async_copy(
                        bufs[b2], acc_sp.at[dst_v.at[j - DEPTH]],
                        ss[b2]).wait()

                @pl.when(j + DEPTH < NCH_PH)
                def _():
                    pltpu.async_copy(hws_hbm.at[src_v.at[j + DEPTH]],
                                     bufs[b2], sg[b2])

                pltpu.make_async_copy(hws_hbm.at[src_v.at[j]], bufs[b],
                                      sg[b]).wait()
                pltpu.async_copy(bufs[b], acc_sp.at[dst_v.at[j]], ss[b],
                                 add=True)

        for j in range(NCH_PH - DEPTH, NCH_PH):
            pltpu.make_async_copy(bufs[j % NBUF], acc_sp.at[dst_v.at[j]],
                                  ss[j % NBUF]).wait()

    plsc.subcore_barrier()
    pltpu.async_copy(acc_sp.at[pl.ds(base, STRIPE)],
                     out_hbm.at[pl.ds(base, STRIPE), c], saux).wait()


def _pre_body(cnt_ref, x_ref, w1_ref, b1_ref, wc1_ref, dinv_ref, hws_ref):
    deg = 1.0 + jnp.sum(cnt_ref[...], axis=0)
    dinv = lax.rsqrt(deg)
    dinv_ref[...] = dinv
    h = jnp.maximum(
        jnp.dot(x_ref[...], w1_ref[...], preferred_element_type=jnp.float32)
        + b1_ref[...][None, :], 0.0)
    hw1 = jnp.dot(h, wc1_ref[...], preferred_element_type=jnp.float32)
    hws_ref[...] = hw1 * dinv[:, None]


def _fused2_body(p_ref, hws_ref, dinv_ref, bc1_ref, w2_ref, b2_ref, wc2_ref,
                 out_ref):
    dinv = dinv_ref[...]
    agg = p_ref[...] + hws_ref[...]
    h1 = jnp.maximum(agg * dinv[:, None] + bc1_ref[...][None, :], 0.0)
    h2 = jnp.maximum(
        jnp.dot(h1, w2_ref[...], preferred_element_type=jnp.float32)
        + b2_ref[...][None, :], 0.0)
    hw2 = jnp.dot(h2, wc2_ref[...], preferred_element_type=jnp.float32)
    out_ref[...] = hw2 * dinv[:, None]


def _fused3_body(p_ref, hws_ref, dinv_ref, bc2_ref, wf_ref, bf_ref, batch_ref,
                 out_ref):
    dinv = dinv_ref[...]
    agg = p_ref[...] + hws_ref[...]
    h3 = jnp.maximum(agg * dinv[:, None] + bc2_ref[...][None, :], 0.0)
    o = jnp.dot(h3, wf_ref[...], preferred_element_type=jnp.float32) \
        + bf_ref[...][None, :]
    seg = batch_ref[...]
    onehot = (seg[:, None]
              == lax.broadcasted_iota(jnp.int32, (NPAD, G), 1)
              ).astype(jnp.float32)
    sums = lax.dot_general(onehot, o, (((0,), (0,)), ((), ())),
                           preferred_element_type=jnp.float32)
    cnts = jnp.sum(onehot, axis=0)
    out_ref[...] = sums / jnp.maximum(cnts, 1.0)[:, None]


def kernel(x, edge_index, batch, W_fc1, b_fc1, W_c1, b_c1,
           W_fc2, b_fc2, W_c2, b_c2, W_fc, b_fc):
    f32 = jnp.float32

    xp = jnp.pad(x, ((0, NPAD - N), (0, 0)))
    batch_p = jnp.pad(batch, (0, NPAD - N), constant_values=G)
    zeros2d = jnp.zeros((NPAD, HH), dtype=f32)

    pad_per_tile = EPT - E // NSUB
    pad_idx = (jnp.arange(pad_per_tile, dtype=jnp.int32) % 16) + N
    pad_blk = jnp.broadcast_to(pad_idx, (NSUB, pad_per_tile))
    src_t = jnp.concatenate(
        [edge_index[0].reshape(NSUB, E // NSUB), pad_blk], axis=1
    ).reshape(NSUB, NCHUNK, CHUNK)
    dst_t = jnp.concatenate(
        [edge_index[1].reshape(NSUB, E // NSUB), pad_blk], axis=1
    ).reshape(NSUB, NCHUNK, CHUNK)
    src_tc = jnp.stack([2 * src_t, 2 * src_t + 1])

    cnt_parts = _sc_count(dst_t)

    dinv, hws1 = pl.pallas_call(
        _pre_body,
        out_shape=(jax.ShapeDtypeStruct((NPAD,), f32),
                   jax.ShapeDtypeStruct((NPAD, H), f32)),
    )(cnt_parts, xp, W_fc1, b_fc1, W_c1)

    parts1 = _sc_agg(hws1.reshape(2 * NPAD, HH), src_tc, dst_t, zeros2d)

    hws2 = pl.pallas_call(
        _fused2_body,
        out_shape=jax.ShapeDtypeStruct((NPAD, H), f32),
    )(parts1.reshape(NPAD, H), hws1, dinv, b_c1, W_fc2, b_fc2, W_c2)

    parts2 = _sc_agg(hws2.reshape(2 * NPAD, HH), src_tc, dst_t, zeros2d)

    out = pl.pallas_call(
        _fused3_body,
        out_shape=jax.ShapeDtypeStruct((G, OUT), f32),
    )(parts2.reshape(NPAD, H), hws2, dinv, b_c2, W_fc, b_fc, batch_p)

    return out


# speedup vs baseline: 1.2710x; 1.2710x over previous
import functools

import jax
import jax.numpy as jnp
from jax import lax
from jax.experimental import pallas as pl
from jax.experimental.pallas import tpu as pltpu
from jax.experimental.pallas import tpu_sc as plsc

N = 10000
E = 320000
F_IN = 128
H = 128
HH = H // 2
OUT = 64
G = 64

NPAD = 10240
NSUB = 16
CHUNK = 128
NCHUNK = 160
EPT = CHUNK * NCHUNK
EPAD = NSUB * EPT
STRIPE = NPAD // NSUB

_mesh = plsc.VectorSubcoreMesh(core_axis_name="c", subcore_axis_name="s")
_sc_params = pltpu.CompilerParams(needs_layout_passes=False)
_sc_agg_params = pltpu.CompilerParams(use_tc_tiling_on_sc=False)


@functools.partial(
    pl.kernel,
    mesh=_mesh,
    out_type=jax.ShapeDtypeStruct((2 * NSUB, NPAD), jnp.float32),
    scratch_types=[
        pltpu.VMEM((NCHUNK // 2, CHUNK), jnp.int32),
        pltpu.VMEM((NPAD,), jnp.float32),
        pltpu.SemaphoreType.DMA,
    ],
    compiler_params=_sc_params,
)
def _sc_count(dst_hbm, out_hbm, idx_v, cnt_v, sem):
    c = lax.axis_index("c")
    s = lax.axis_index("s")
    wid = s * 2 + c
    pltpu.async_copy(dst_hbm.at[s, pl.ds(c * (NCHUNK // 2), NCHUNK // 2)],
                     idx_v, sem).wait()

    @pl.loop(0, NPAD, step=16)
    def _(i):
        cnt_v[pl.ds(i, 16)] = jnp.zeros((16,), jnp.float32)

    ones = jnp.ones((16,), jnp.float32)

    @pl.loop(0, NCHUNK // 2)
    def _(j):
        @pl.loop(0, CHUNK, step=16)
        def _(l):
            idx16 = idx_v[j, pl.ds(l, 16)]
            plsc.addupdate_scatter(cnt_v, [idx16], ones)

    pltpu.async_copy(cnt_v, out_hbm.at[wid], sem).wait()


NBUF = 8
DEPTH = NBUF // 2
NPH = 2
NCH_PH = NCHUNK // NPH


@functools.partial(
    pl.kernel,
    mesh=_mesh,
    out_type=jax.ShapeDtypeStruct((NPAD, H), jnp.float32),
    scratch_types=[
        pltpu.VMEM((NCH_PH, CHUNK), jnp.int32),
        pltpu.VMEM((NCH_PH, CHUNK), jnp.int32),
    ] + [pltpu.VMEM((CHUNK, HH), jnp.float32) for _ in range(NBUF)] + [
        pltpu.VMEM_SHARED((NPAD, HH), jnp.float32),
    ] + [pltpu.SemaphoreType.DMA for _ in range(2 * NBUF + 1)],
    compiler_params=_sc_agg_params,
)
def _sc_agg(hws_hbm, src_hbm, dst_hbm, zero_hbm, out_hbm,
            src_v, dst_v, *rest):
    bufs = rest[:NBUF]
    acc_sp = rest[NBUF]
    sg = rest[NBUF + 1:2 * NBUF + 1]
    ss = rest[2 * NBUF + 1:3 * NBUF + 1]
    saux = rest[3 * NBUF + 1]

    c = lax.axis_index("c")
    s = lax.axis_index("s")
    base = s * STRIPE

    pltpu.async_copy(zero_hbm.at[pl.ds(base, STRIPE)],
                     acc_sp.at[pl.ds(base, STRIPE)], saux).wait()
    plsc.subcore_barrier()

    @pl.loop(0, NPH)
    def _(p):
        off = p * NCH_PH
        cp_src = pltpu.async_copy(
            src_hbm.at[c, s, pl.ds(off, NCH_PH)], src_v, sg[0])
        cp_dst = pltpu.async_copy(
            dst_hbm.at[s, pl.ds(off, NCH_PH)], dst_v, sg[1])
        cp_src.wait()
        cp_dst.wait()

        for j in range(DEPTH):
            pltpu.async_copy(hws_hbm.at[src_v.at[j]], bufs[j], sg[j])

        @pl.loop(0, NCH_PH // NBUF)
        def _(k):
            j0 = NBUF * k
            for b in range(NBUF):
                j = j0 + b
                b2 = (b + DEPTH) % NBUF

                @pl.when(j >= DEPTH)
                def _():
                    pltpu.make_async_copy(
                        bufs[b2], acc_sp.at[dst_v.at[j - DEPTH]],
                        ss[b2]).wait()

                @pl.when(j + DEPTH < NCH_PH)
                def _():
                    pltpu.async_copy(hws_hbm.at[src_v.at[j + DEPTH]],
                                     bufs[b2], sg[b2])

                pltpu.make_async_copy(hws_hbm.at[src_v.at[j]], bufs[b],
                                      sg[b]).wait()
                pltpu.async_copy(bufs[b], acc_sp.at[dst_v.at[j]], ss[b],
                                 add=True)

        for j in range(NCH_PH - DEPTH, NCH_PH):
            pltpu.make_async_copy(bufs[j % NBUF], acc_sp.at[dst_v.at[j]],
                                  ss[j % NBUF]).wait()

    plsc.subcore_barrier()
    pltpu.async_copy(acc_sp.at[pl.ds(base, STRIPE)],
                     out_hbm.at[pl.ds(base, STRIPE), pl.ds(c * HH, HH)],
                     saux).wait()


def _pre_body(cnt_ref, x_ref, w1_ref, b1_ref, wc1_ref, dinv_ref, hws_ref):
    deg = 1.0 + jnp.sum(cnt_ref[...], axis=0)
    dinv = lax.rsqrt(deg)
    dinv_ref[...] = dinv
    h = jnp.maximum(
        jnp.dot(x_ref[...], w1_ref[...], preferred_element_type=jnp.float32)
        + b1_ref[...][None, :], 0.0)
    hw1 = jnp.dot(h, wc1_ref[...], preferred_element_type=jnp.float32)
    hws_ref[...] = hw1 * dinv[:, None]


def _fused2_body(p_ref, hws_ref, dinv_ref, bc1_ref, w2_ref, b2_ref, wc2_ref,
                 out_ref):
    dinv = dinv_ref[...]
    agg = p_ref[...] + hws_ref[...]
    h1 = jnp.maximum(agg * dinv[:, None] + bc1_ref[...][None, :], 0.0)
    h2 = jnp.maximum(
        jnp.dot(h1, w2_ref[...], preferred_element_type=jnp.float32)
        + b2_ref[...][None, :], 0.0)
    hw2 = jnp.dot(h2, wc2_ref[...], preferred_element_type=jnp.float32)
    out_ref[...] = hw2 * dinv[:, None]


def _fused3_body(p_ref, hws_ref, dinv_ref, bc2_ref, wf_ref, bf_ref, batch_ref,
                 out_ref):
    dinv = dinv_ref[...]
    agg = p_ref[...] + hws_ref[...]
    h3 = jnp.maximum(agg * dinv[:, None] + bc2_ref[...][None, :], 0.0)
    o = jnp.dot(h3, wf_ref[...], preferred_element_type=jnp.float32) \
        + bf_ref[...][None, :]
    seg = batch_ref[...]
    onehot = (seg[:, None]
              == lax.broadcasted_iota(jnp.int32, (NPAD, G), 1)
              ).astype(jnp.float32)
    sums = lax.dot_general(onehot, o, (((0,), (0,)), ((), ())),
                           preferred_element_type=jnp.float32)
    cnts = jnp.sum(onehot, axis=0)
    out_ref[...] = sums / jnp.maximum(cnts, 1.0)[:, None]


def kernel(x, edge_index, batch, W_fc1, b_fc1, W_c1, b_c1,
           W_fc2, b_fc2, W_c2, b_c2, W_fc, b_fc):
    f32 = jnp.float32

    xp = jnp.pad(x, ((0, NPAD - N), (0, 0)))
    batch_p = jnp.pad(batch, (0, NPAD - N), constant_values=G)
    zeros2d = jnp.zeros((NPAD, HH), dtype=f32)

    pad_per_tile = EPT - E // NSUB
    pad_idx = (jnp.arange(pad_per_tile, dtype=jnp.int32) % 16) + N
    pad_blk = jnp.broadcast_to(pad_idx, (NSUB, pad_per_tile))
    src_t = jnp.concatenate(
        [edge_index[0].reshape(NSUB, E // NSUB), pad_blk], axis=1
    ).reshape(NSUB, NCHUNK, CHUNK)
    dst_t = jnp.concatenate(
        [edge_index[1].reshape(NSUB, E // NSUB), pad_blk], axis=1
    ).reshape(NSUB, NCHUNK, CHUNK)
    src_tc = jnp.stack([2 * src_t, 2 * src_t + 1])

    cnt_parts = _sc_count(dst_t)

    dinv, hws1 = pl.pallas_call(
        _pre_body,
        out_shape=(jax.ShapeDtypeStruct((NPAD,), f32),
                   jax.ShapeDtypeStruct((NPAD, H), f32)),
    )(cnt_parts, xp, W_fc1, b_fc1, W_c1)

    parts1 = _sc_agg(hws1.reshape(2 * NPAD, HH), src_tc, dst_t, zeros2d)

    hws2 = pl.pallas_call(
        _fused2_body,
        out_shape=jax.ShapeDtypeStruct((NPAD, H), f32),
    )(parts1, hws1, dinv, b_c1, W_fc2, b_fc2, W_c2)

    parts2 = _sc_agg(hws2.reshape(2 * NPAD, HH), src_tc, dst_t, zeros2d)

    out = pl.pallas_call(
        _fused3_body,
        out_shape=jax.ShapeDtypeStruct((G, OUT), f32),
    )(parts2, hws2, dinv, b_c2, W_fc, b_fc, batch_p)

    return out
